# baseline (speedup 1.0000x reference)
import functools

import jax
import jax.numpy as jnp
from jax import lax
from jax.experimental import pallas as pl
from jax.experimental.pallas import tpu as pltpu
from jax.experimental.pallas import tpu_sc as plsc

N = 10000
E = 320000
D = 128
B = 64
OUT = 1

NC = 2
NS = 16
NW = NC * NS
EPW = E // NW
CH = 80
CPW = EPW // CH
NBUF = 3
NP = 10112
RPS = NP // NS
LRPS = N - (NS - 1) * RPS

BN = 1000
GN = N // BN


def _sc_aggregate(h, pk3, zeros):
    mesh = plsc.VectorSubcoreMesh(core_axis_name="c", subcore_axis_name="s")

    @functools.partial(
        pl.kernel,
        out_type=jax.ShapeDtypeStruct((NC, NP, D), jnp.float32),
        mesh=mesh,
        scratch_types=[
            pltpu.VMEM_SHARED((NP, D), jnp.float32),
            pltpu.VMEM((CPW, CH), jnp.int32),
            pltpu.VMEM((NBUF, CH), jnp.int32),
            pltpu.VMEM((NBUF, CH), jnp.int32),
            pltpu.VMEM((NBUF, CH, D), jnp.float32),
            pltpu.SemaphoreType.DMA,
            pltpu.SemaphoreType.DMA,
            pltpu.SemaphoreType.DMA,
        ],
    )
    def agg(h_hbm, pk_hbm, z_hbm, out_hbm, acc, pkv, srcu, dstu, rows, sem,
            sems, semi):
        c = lax.axis_index("c")
        s = lax.axis_index("s")
        w = c * NS + s
        pltpu.async_copy(pk_hbm.at[w], pkv, semi)
        base = s * RPS
        @pl.when(s < NS - 1)
        def _():
            @pl.when(c == 0)
            def _():
                pltpu.sync_copy(h_hbm.at[pl.ds(base, RPS)],
                                acc.at[pl.ds(base, RPS)])
            @pl.when(c != 0)
            def _():
                pltpu.sync_copy(z_hbm, acc.at[pl.ds(base, RPS)])
        @pl.when(s == NS - 1)
        def _():
            @pl.when(c == 0)
            def _():
                pltpu.sync_copy(h_hbm.at[pl.ds(base, LRPS)],
                                acc.at[pl.ds(base, LRPS)])
            @pl.when(c != 0)
            def _():
                pltpu.sync_copy(z_hbm.at[pl.ds(0, LRPS)],
                                acc.at[pl.ds(base, LRPS)])
        pltpu.make_async_copy(pk_hbm.at[w], pkv, semi).wait()

        def unpack(j, b):
            for l in range(CH // 16):
                pk = pkv[j, pl.ds(16 * l, 16)]
                srcu[b, pl.ds(16 * l, 16)] = pk & 0xFFFF
                dstu[b, pl.ds(16 * l, 16)] = lax.shift_right_logical(pk, 16)

        def scatter_drain():
            pltpu.make_async_copy(
                h_hbm.at[srcu.at[0]], rows.at[0], sems).wait()

        for b in range(2):
            unpack(b, b)
            pltpu.async_copy(h_hbm.at[srcu.at[b]], rows.at[b], sem)
        plsc.subcore_barrier()

        def step(j, b):
            pltpu.make_async_copy(
                h_hbm.at[srcu.at[b]], rows.at[b], sem).wait()

            @pl.when(j >= 1)
            def _():
                scatter_drain()
            pltpu.async_copy(rows.at[b], acc.at[dstu.at[b]], sems, add=True)

            @pl.when(j + 2 < CPW)
            def _():
                unpack(j + 2, (b + 2) % NBUF)
                pltpu.async_copy(h_hbm.at[srcu.at[(b + 2) % NBUF]],
                                 rows.at[(b + 2) % NBUF], sem)

        def body(i, carry):
            for b in range(NBUF):
                step(i * NBUF + b, b)
            return carry

        lax.fori_loop(0, CPW // NBUF, body, 0, unroll=False)
        for t in range(CPW - (CPW // NBUF) * NBUF):
            step((CPW // NBUF) * NBUF + t, t)
        scatter_drain()
        plsc.subcore_barrier()
        @pl.when(s < NS - 1)
        def _():
            pltpu.sync_copy(acc.at[pl.ds(base, RPS)],
                            out_hbm.at[c, pl.ds(base, RPS)])
        @pl.when(s == NS - 1)
        def _():
            pltpu.sync_copy(acc.at[pl.ds(base, LRPS)],
                            out_hbm.at[c, pl.ds(base, LRPS)])

    return agg(h, pk3, zeros)


def _mlp_body(agg, w1_ref, b1_ref, w2_ref, b2_ref):
    h1 = jnp.maximum(
        jnp.dot(agg, w1_ref[...], preferred_element_type=jnp.float32)
        + b1_ref[...], 0.0)
    return (jnp.dot(h1, w2_ref[...], preferred_element_type=jnp.float32)
            + b2_ref[...])


def _tc_mlp1(p, w1t, b1, w2t, b2):
    def body(p_ref, w1_ref, b1_ref, w2_ref, b2_ref, o_ref):
        agg = p_ref[0] + p_ref[1]
        o_ref[...] = _mlp_body(agg, w1_ref, b1_ref, w2_ref, b2_ref)

    return pl.pallas_call(
        body,
        grid=(GN,),
        in_specs=[
            pl.BlockSpec((NC, BN, D), lambda i: (0, i, 0)),
            pl.BlockSpec((D, D), lambda i: (0, 0)),
            pl.BlockSpec((1, D), lambda i: (0, 0)),
            pl.BlockSpec((D, D), lambda i: (0, 0)),
            pl.BlockSpec((1, D), lambda i: (0, 0)),
        ],
        out_specs=pl.BlockSpec((BN, D), lambda i: (i, 0)),
        out_shape=jax.ShapeDtypeStruct((N, D), jnp.float32),
    )(p, w1t, b1, w2t, b2)


def _tc_mlp2_pool_reg(q, xb3, w1t, b1, w2t, b2, wr1t, br1, wr2t, br2):
    def body(q_ref, xb_ref, w1_ref, b1_ref, w2_ref, b2_ref,
             wr1_ref, br1_ref, wr2_ref, br2_ref, o_ref, pooled, cnt):
        i = pl.program_id(0)

        @pl.when(i == 0)
        def _():
            pooled[...] = jnp.zeros_like(pooled)
            cnt[...] = jnp.zeros_like(cnt)

        agg = q_ref[0] + q_ref[1]
        h2 = _mlp_body(agg, w1_ref, b1_ref, w2_ref, b2_ref)
        gid = lax.broadcasted_iota(jnp.int32, (B, BN), 0)
        maskf = (jnp.broadcast_to(xb_ref[0], (B, BN)) == gid
                 ).astype(jnp.float32)
        pooled[...] += jnp.dot(maskf, h2, preferred_element_type=jnp.float32)
        cnt[...] += jnp.broadcast_to(
            jnp.sum(maskf, axis=1, keepdims=True), (B, D))

        @pl.when(i == GN - 1)
        def _():
            mean = pooled[...] / jnp.maximum(cnt[...], 1.0)
            r = jnp.maximum(
                jnp.dot(mean, wr1_ref[...], preferred_element_type=jnp.float32)
                + br1_ref[...], 0.0)
            o_ref[...] = (jnp.dot(r, wr2_ref[...],
                                  preferred_element_type=jnp.float32)
                          + br2_ref[...])

    return pl.pallas_call(
        body,
        grid=(GN,),
        in_specs=[
            pl.BlockSpec((NC, BN, D), lambda i: (0, i, 0)),
            pl.BlockSpec((1, 1, BN), lambda i: (i, 0, 0)),
            pl.BlockSpec((D, D), lambda i: (0, 0)),
            pl.BlockSpec((1, D), lambda i: (0, 0)),
            pl.BlockSpec((D, D), lambda i: (0, 0)),
            pl.BlockSpec((1, D), lambda i: (0, 0)),
            pl.BlockSpec((D, D), lambda i: (0, 0)),
            pl.BlockSpec((1, D), lambda i: (0, 0)),
            pl.BlockSpec((D, D), lambda i: (0, 0)),
            pl.BlockSpec((1, D), lambda i: (0, 0)),
        ],
        out_specs=pl.BlockSpec((B, D), lambda i: (0, 0)),
        out_shape=jax.ShapeDtypeStruct((B, D), jnp.float32),
        scratch_shapes=[
            pltpu.VMEM((B, D), jnp.float32),
            pltpu.VMEM((B, D), jnp.float32),
        ],
    )(q, xb3, w1t, b1, w2t, b2, wr1t, br1, wr2t, br2)


def kernel(x, edge_index, pos, x_batch, W1a, b1a, W2a, b2a,
           W1b, b1b, W2b, b2b, Wr1, br1, Wr2, br2):
    pk3 = (edge_index[0] | (edge_index[1] << 16)).reshape(NW, CPW, CH)
    zeros = jnp.zeros((RPS, D), jnp.float32)
    xb3 = x_batch.reshape(GN, 1, BN)

    w1at, w2at = W1a.T, W2a.T
    w1bt, w2bt = W1b.T, W2b.T
    wr1t = Wr1.T
    wr2t = jnp.pad(Wr2.T, ((0, 0), (0, D - OUT)))
    b1a2, b2a2 = b1a.reshape(1, D), b2a.reshape(1, D)
    b1b2, b2b2 = b1b.reshape(1, D), b2b.reshape(1, D)
    br12 = br1.reshape(1, D)
    br22 = jnp.pad(br2.reshape(1, OUT), ((0, 0), (0, D - OUT)))

    p = _sc_aggregate(x, pk3, zeros)
    h = _tc_mlp1(p, w1at, b1a2, w2at, b2a2)
    q = _sc_aggregate(h, pk3, zeros)
    out = _tc_mlp2_pool_reg(q, xb3, w1bt, b1b2, w2bt, b2b2,
                            wr1t, br12, wr2t, br22)
    return out[:, :OUT]


# speedup vs baseline: 1.0419x; 1.0419x over previous
import functools

import jax
import jax.numpy as jnp
from jax import lax
from jax.experimental import pallas as pl
from jax.experimental.pallas import tpu as pltpu
from jax.experimental.pallas import tpu_sc as plsc

N = 10000
E = 320000
D = 128
B = 64
OUT = 1

NC = 2
NS = 16
NW = NC * NS
EPW = E // NW
CH = 80
CPW = EPW // CH
NBUF = 3
NP = 10112
RPS = NP // NS
LRPS = N - (NS - 1) * RPS

BN = 1000
GN = N // BN


def _sc_aggregate(h, pk3, zeros):
    mesh = plsc.VectorSubcoreMesh(core_axis_name="c", subcore_axis_name="s")

    @functools.partial(
        pl.kernel,
        out_type=jax.ShapeDtypeStruct((NC, NP, D), jnp.float32),
        mesh=mesh,
        scratch_types=[
            pltpu.VMEM_SHARED((NP, D), jnp.float32),
            pltpu.VMEM((CPW, CH), jnp.int32),
            pltpu.VMEM((NBUF, CH), jnp.int32),
            pltpu.VMEM((NBUF, CH), jnp.int32),
            pltpu.VMEM((NBUF, CH, D), jnp.float32),
            pltpu.SemaphoreType.DMA,
            pltpu.SemaphoreType.DMA,
        ],
    )
    def agg(h_hbm, pk_hbm, z_hbm, out_hbm, acc, pkv, srcu, dstu, rows, sem,
            semi):
        c = lax.axis_index("c")
        s = lax.axis_index("s")
        w = c * NS + s
        pltpu.async_copy(pk_hbm.at[w], pkv, semi)
        base = s * RPS
        @pl.when(s < NS - 1)
        def _():
            @pl.when(c == 0)
            def _():
                pltpu.sync_copy(h_hbm.at[pl.ds(base, RPS)],
                                acc.at[pl.ds(base, RPS)])
            @pl.when(c != 0)
            def _():
                pltpu.sync_copy(z_hbm, acc.at[pl.ds(base, RPS)])
        @pl.when(s == NS - 1)
        def _():
            @pl.when(c == 0)
            def _():
                pltpu.sync_copy(h_hbm.at[pl.ds(base, LRPS)],
                                acc.at[pl.ds(base, LRPS)])
            @pl.when(c != 0)
            def _():
                pltpu.sync_copy(z_hbm.at[pl.ds(0, LRPS)],
                                acc.at[pl.ds(base, LRPS)])
        pltpu.make_async_copy(pk_hbm.at[w], pkv, semi).wait()

        def unpack(j, b):
            for l in range(CH // 16):
                pk = pkv[j, pl.ds(16 * l, 16)]
                srcu[b, pl.ds(16 * l, 16)] = pk & 0xFFFF
                dstu[b, pl.ds(16 * l, 16)] = lax.shift_right_logical(pk, 16)

        for b in range(NBUF):
            unpack(b, b)
            pltpu.async_copy(h_hbm.at[srcu.at[b]], rows.at[b], sem)
        plsc.subcore_barrier()

        def body(i, carry):
            for b in range(NBUF):
                j = i * NBUF + b
                pltpu.make_async_copy(
                    h_hbm.at[srcu.at[b]], rows.at[b], sem).wait()
                pltpu.sync_copy(rows.at[b], acc.at[dstu.at[b]], add=True)

                @pl.when(j + NBUF < CPW)
                def _():
                    unpack(j + NBUF, b)
                    pltpu.async_copy(h_hbm.at[srcu.at[b]], rows.at[b], sem)
            return carry

        lax.fori_loop(0, CPW // NBUF, body, 0, unroll=False)
        for t in range(CPW - (CPW // NBUF) * NBUF):
            pltpu.make_async_copy(
                h_hbm.at[srcu.at[t]], rows.at[t], sem).wait()
            pltpu.sync_copy(rows.at[t], acc.at[dstu.at[t]], add=True)
        plsc.subcore_barrier()
        @pl.when(s < NS - 1)
        def _():
            pltpu.sync_copy(acc.at[pl.ds(base, RPS)],
                            out_hbm.at[c, pl.ds(base, RPS)])
        @pl.when(s == NS - 1)
        def _():
            pltpu.sync_copy(acc.at[pl.ds(base, LRPS)],
                            out_hbm.at[c, pl.ds(base, LRPS)])

    return agg(h, pk3, zeros)


def _mlp_body(agg, w1_ref, b1_ref, w2_ref, b2_ref):
    h1 = jnp.maximum(
        jnp.dot(agg, w1_ref[...], preferred_element_type=jnp.float32)
        + b1_ref[...], 0.0)
    return (jnp.dot(h1, w2_ref[...], preferred_element_type=jnp.float32)
            + b2_ref[...])


def _tc_mlp1(p, w1t, b1, w2t, b2):
    def body(p_ref, w1_ref, b1_ref, w2_ref, b2_ref, o_ref):
        agg = p_ref[0] + p_ref[1]
        o_ref[...] = _mlp_body(agg, w1_ref, b1_ref, w2_ref, b2_ref)

    return pl.pallas_call(
        body,
        grid=(GN,),
        in_specs=[
            pl.BlockSpec((NC, BN, D), lambda i: (0, i, 0)),
            pl.BlockSpec((D, D), lambda i: (0, 0)),
            pl.BlockSpec((1, D), lambda i: (0, 0)),
            pl.BlockSpec((D, D), lambda i: (0, 0)),
            pl.BlockSpec((1, D), lambda i: (0, 0)),
        ],
        out_specs=pl.BlockSpec((BN, D), lambda i: (i, 0)),
        out_shape=jax.ShapeDtypeStruct((N, D), jnp.float32),
    )(p, w1t, b1, w2t, b2)


def _tc_mlp2_pool_reg(q, xb3, w1t, b1, w2t, b2, wr1t, br1, wr2t, br2):
    def body(q_ref, xb_ref, w1_ref, b1_ref, w2_ref, b2_ref,
             wr1_ref, br1_ref, wr2_ref, br2_ref, o_ref, pooled, cnt):
        i = pl.program_id(0)

        @pl.when(i == 0)
        def _():
            pooled[...] = jnp.zeros_like(pooled)
            cnt[...] = jnp.zeros_like(cnt)

        agg = q_ref[0] + q_ref[1]
        h2 = _mlp_body(agg, w1_ref, b1_ref, w2_ref, b2_ref)
        gid = lax.broadcasted_iota(jnp.int32, (B, BN), 0)
        maskf = (jnp.broadcast_to(xb_ref[0], (B, BN)) == gid
                 ).astype(jnp.float32)
        pooled[...] += jnp.dot(maskf, h2, preferred_element_type=jnp.float32)
        cnt[...] += jnp.broadcast_to(
            jnp.sum(maskf, axis=1, keepdims=True), (B, D))

        @pl.when(i == GN - 1)
        def _():
            mean = pooled[...] / jnp.maximum(cnt[...], 1.0)
            r = jnp.maximum(
                jnp.dot(mean, wr1_ref[...], preferred_element_type=jnp.float32)
                + br1_ref[...], 0.0)
            o_ref[...] = (jnp.dot(r, wr2_ref[...],
                                  preferred_element_type=jnp.float32)
                          + br2_ref[...])

    return pl.pallas_call(
        body,
        grid=(GN,),
        in_specs=[
            pl.BlockSpec((NC, BN, D), lambda i: (0, i, 0)),
            pl.BlockSpec((1, 1, BN), lambda i: (i, 0, 0)),
            pl.BlockSpec((D, D), lambda i: (0, 0)),
            pl.BlockSpec((1, D), lambda i: (0, 0)),
            pl.BlockSpec((D, D), lambda i: (0, 0)),
            pl.BlockSpec((1, D), lambda i: (0, 0)),
            pl.BlockSpec((D, D), lambda i: (0, 0)),
            pl.BlockSpec((1, D), lambda i: (0, 0)),
            pl.BlockSpec((D, D), lambda i: (0, 0)),
            pl.BlockSpec((1, D), lambda i: (0, 0)),
        ],
        out_specs=pl.BlockSpec((B, D), lambda i: (0, 0)),
        out_shape=jax.ShapeDtypeStruct((B, D), jnp.float32),
        scratch_shapes=[
            pltpu.VMEM((B, D), jnp.float32),
            pltpu.VMEM((B, D), jnp.float32),
        ],
    )(q, xb3, w1t, b1, w2t, b2, wr1t, br1, wr2t, br2)


def kernel(x, edge_index, pos, x_batch, W1a, b1a, W2a, b2a,
           W1b, b1b, W2b, b2b, Wr1, br1, Wr2, br2):
    pk3 = (edge_index[0] | (edge_index[1] << 16)).reshape(NW, CPW, CH)
    zeros = jnp.zeros((RPS, D), jnp.float32)
    xb3 = x_batch.reshape(GN, 1, BN)

    w1at, w2at = W1a.T, W2a.T
    w1bt, w2bt = W1b.T, W2b.T
    wr1t = Wr1.T
    wr2t = jnp.pad(Wr2.T, ((0, 0), (0, D - OUT)))
    b1a2, b2a2 = b1a.reshape(1, D), b2a.reshape(1, D)
    b1b2, b2b2 = b1b.reshape(1, D), b2b.reshape(1, D)
    br12 = br1.reshape(1, D)
    br22 = jnp.pad(br2.reshape(1, OUT), ((0, 0), (0, D - OUT)))

    p = _sc_aggregate(x, pk3, zeros)
    h = _tc_mlp1(p, w1at, b1a2, w2at, b2a2)
    q = _sc_aggregate(h, pk3, zeros)
    out = _tc_mlp2_pool_reg(q, xb3, w1bt, b1b2, w2bt, b2b2,
                            wr1t, br12, wr2t, br22)
    return out[:, :OUT]
